# Initial kernel scaffold; baseline (speedup 1.0000x reference)
import jax
import jax.numpy as jnp
from jax.experimental import pallas as pl


def kernel(input_ids, token_type_ids, word_emb, type_emb, pos_emb, gamma, beta):
    raise NotImplementedError("write your pallas kernel here")


# speedup vs baseline: 2.9026x; 2.9026x over previous
import functools

import jax
import jax.numpy as jnp
from jax import lax
from jax.experimental import pallas as pl
from jax.experimental.pallas import tpu as pltpu
from jax.experimental.pallas import tpu_sc as plsc

VOCAB = 30522
D = 768
B = 128
S = 512
TOKENS = B * S
EPS = 1e-12

NW = 32
TOK_PER_W = TOKENS // NW
CHUNK = 64
NCHUNK = TOK_PER_W // CHUNK


def _sc_gather(ids3, word_emb):
    mesh = plsc.VectorSubcoreMesh(core_axis_name="c", subcore_axis_name="s")

    @functools.partial(
        pl.kernel,
        mesh=mesh,
        out_type=jax.ShapeDtypeStruct((TOKENS, D), jnp.float32),
        scratch_types=[
            pltpu.VMEM((NCHUNK, CHUNK), jnp.int32),
            pltpu.VMEM((CHUNK, D), jnp.float32),
            pltpu.VMEM((CHUNK, D), jnp.float32),
            pltpu.SemaphoreType.DMA,
            pltpu.SemaphoreType.DMA,
            pltpu.SemaphoreType.DMA,
            pltpu.SemaphoreType.DMA,
        ],
    )
    def k(ids_hbm, table_hbm, out_hbm, idx_v, buf0, buf1, g0, g1, o0, o1):
        cid = lax.axis_index("c")
        sid = lax.axis_index("s")
        wid = sid * 2 + cid
        base = wid * TOK_PER_W
        pltpu.sync_copy(ids_hbm.at[wid], idx_v)

        def gather(c, buf, sem):
            return pltpu.async_copy(table_hbm.at[idx_v.at[c]], buf, sem)

        def put(c, buf, sem):
            return pltpu.async_copy(
                buf, out_hbm.at[pl.ds(base + c * CHUNK, CHUNK)], sem)

        gather(0, buf0, g0)
        gather(1, buf1, g1)

        def body(i, _):
            c0 = 2 * i
            c1 = c0 + 1
            pltpu.make_async_copy(table_hbm.at[idx_v.at[c0]], buf0, g0).wait()
            put(c0, buf0, o0)
            pltpu.make_async_copy(table_hbm.at[idx_v.at[c1]], buf1, g1).wait()
            put(c1, buf1, o1)

            @pl.when(i + 1 < NCHUNK // 2)
            def _():
                pltpu.make_async_copy(
                    buf0, out_hbm.at[pl.ds(base + c0 * CHUNK, CHUNK)],
                    o0).wait()
                gather(c0 + 2, buf0, g0)
                pltpu.make_async_copy(
                    buf1, out_hbm.at[pl.ds(base + c1 * CHUNK, CHUNK)],
                    o1).wait()
                gather(c1 + 2, buf1, g1)
            return 0

        lax.fori_loop(0, NCHUNK // 2, body, 0)
        last0 = NCHUNK - 2
        last1 = NCHUNK - 1
        pltpu.make_async_copy(
            buf0, out_hbm.at[pl.ds(base + last0 * CHUNK, CHUNK)], o0).wait()
        pltpu.make_async_copy(
            buf1, out_hbm.at[pl.ds(base + last1 * CHUNK, CHUNK)], o1).wait()

    return k(ids3, word_emb)


def _ln_body(w_ref, tt_ref, te_ref, pe_ref, g_ref, b_ref, o_ref):
    x = w_ref[0]
    t = tt_ref[0]
    e0 = te_ref[0:1, :]
    e1 = te_ref[1:2, :]
    x = x + pe_ref[...] + e0 + t * (e1 - e0)
    mean = jnp.mean(x, axis=-1, keepdims=True)
    xc = x - mean
    var = jnp.mean(xc * xc, axis=-1, keepdims=True)
    inv = lax.rsqrt(var + EPS)
    o_ref[0] = xc * inv * g_ref[...] + b_ref[...]


def _tc_ln(w_rows, tt_col, type_emb, pos_emb, gamma, beta):
    wr = w_rows.reshape(B, S, D)
    return pl.pallas_call(
        _ln_body,
        grid=(B,),
        in_specs=[
            pl.BlockSpec((1, S, D), lambda b: (b, 0, 0)),
            pl.BlockSpec((1, S, 1), lambda b: (b, 0, 0)),
            pl.BlockSpec((2, D), lambda b: (0, 0)),
            pl.BlockSpec((S, D), lambda b: (0, 0)),
            pl.BlockSpec((1, D), lambda b: (0, 0)),
            pl.BlockSpec((1, D), lambda b: (0, 0)),
        ],
        out_specs=pl.BlockSpec((1, S, D), lambda b: (b, 0, 0)),
        out_shape=jax.ShapeDtypeStruct((B, S, D), jnp.float32),
    )(wr, tt_col, type_emb, pos_emb, gamma, beta)


def kernel(input_ids, token_type_ids, word_emb, type_emb, pos_emb, gamma, beta):
    ids3 = input_ids.astype(jnp.int32).reshape(NW, NCHUNK, CHUNK)
    w_rows = _sc_gather(ids3, word_emb)
    tt_col = token_type_ids.astype(jnp.float32).reshape(B, S, 1)
    return _tc_ln(w_rows, tt_col, type_emb, pos_emb,
                  gamma.reshape(1, D), beta.reshape(1, D))


# speedup vs baseline: 3.0937x; 1.0658x over previous
import functools

import jax
import jax.numpy as jnp
from jax import lax
from jax.experimental import pallas as pl
from jax.experimental.pallas import tpu as pltpu
from jax.experimental.pallas import tpu_sc as plsc

VOCAB = 30522
D = 768
B = 128
S = 512
EPS = 1e-12

NW = 32
NSPLIT = 4
BSPLIT = B // NSPLIT
TOK_SPLIT = BSPLIT * S
TOK_PER_W = TOK_SPLIT // NW
CHUNK = 64
NCHUNK = TOK_PER_W // CHUNK


def _sc_gather(ids3, word_emb):
    mesh = plsc.VectorSubcoreMesh(core_axis_name="c", subcore_axis_name="s")

    @functools.partial(
        pl.kernel,
        mesh=mesh,
        out_type=jax.ShapeDtypeStruct((TOK_SPLIT, D), jnp.float32),
        scratch_types=[
            pltpu.VMEM((NCHUNK, CHUNK), jnp.int32),
            pltpu.VMEM((CHUNK, D), jnp.float32),
            pltpu.VMEM((CHUNK, D), jnp.float32),
            pltpu.SemaphoreType.DMA,
            pltpu.SemaphoreType.DMA,
            pltpu.SemaphoreType.DMA,
            pltpu.SemaphoreType.DMA,
        ],
    )
    def k(ids_hbm, table_hbm, out_hbm, idx_v, buf0, buf1, g0, g1, o0, o1):
        cid = lax.axis_index("c")
        sid = lax.axis_index("s")
        wid = sid * 2 + cid
        base = wid * TOK_PER_W
        pltpu.sync_copy(ids_hbm.at[wid], idx_v)

        def gather(c, buf, sem):
            return pltpu.async_copy(table_hbm.at[idx_v.at[c]], buf, sem)

        def put(c, buf, sem):
            return pltpu.async_copy(
                buf, out_hbm.at[pl.ds(base + c * CHUNK, CHUNK)], sem)

        gather(0, buf0, g0)
        gather(1, buf1, g1)

        def body(i, _):
            c0 = 2 * i
            c1 = c0 + 1
            pltpu.make_async_copy(table_hbm.at[idx_v.at[c0]], buf0, g0).wait()
            put(c0, buf0, o0)
            pltpu.make_async_copy(table_hbm.at[idx_v.at[c1]], buf1, g1).wait()
            put(c1, buf1, o1)

            @pl.when(i + 1 < NCHUNK // 2)
            def _():
                pltpu.make_async_copy(
                    buf0, out_hbm.at[pl.ds(base + c0 * CHUNK, CHUNK)],
                    o0).wait()
                gather(c0 + 2, buf0, g0)
                pltpu.make_async_copy(
                    buf1, out_hbm.at[pl.ds(base + c1 * CHUNK, CHUNK)],
                    o1).wait()
                gather(c1 + 2, buf1, g1)
            return 0

        lax.fori_loop(0, NCHUNK // 2, body, 0)
        last0 = NCHUNK - 2
        last1 = NCHUNK - 1
        pltpu.make_async_copy(
            buf0, out_hbm.at[pl.ds(base + last0 * CHUNK, CHUNK)], o0).wait()
        pltpu.make_async_copy(
            buf1, out_hbm.at[pl.ds(base + last1 * CHUNK, CHUNK)], o1).wait()

    return k(ids3, word_emb)


def _ln_body_first(w_ref, tt_ref, te_ref, pe_ref, g_ref, b_ref, o_ref):
    x = w_ref[0]
    t = tt_ref[0]
    e0 = te_ref[0:1, :]
    e1 = te_ref[1:2, :]
    x = x + pe_ref[...] + e0 + t * (e1 - e0)
    mean = jnp.mean(x, axis=-1, keepdims=True)
    xc = x - mean
    var = jnp.mean(xc * xc, axis=-1, keepdims=True)
    inv = lax.rsqrt(var + EPS)
    o_ref[0] = xc * inv * g_ref[...] + b_ref[...]


def _ln_body(w_ref, tt_ref, te_ref, pe_ref, g_ref, b_ref, acc_ref, o_ref):
    del acc_ref
    _ln_body_first(w_ref, tt_ref, te_ref, pe_ref, g_ref, b_ref, o_ref)


def _tc_ln_part(w_rows, tt_col, type_emb, pos_emb, gamma, beta, prev, part):
    wr = w_rows.reshape(BSPLIT, S, D)
    off = part * BSPLIT
    in_specs = [
        pl.BlockSpec((1, S, D), lambda b: (b, 0, 0)),
        pl.BlockSpec((1, S, 1), lambda b, off=off: (off + b, 0, 0)),
        pl.BlockSpec((2, D), lambda b: (0, 0)),
        pl.BlockSpec((S, D), lambda b: (0, 0)),
        pl.BlockSpec((1, D), lambda b: (0, 0)),
        pl.BlockSpec((1, D), lambda b: (0, 0)),
    ]
    args = [wr, tt_col, type_emb, pos_emb, gamma, beta]
    if prev is None:
        body = _ln_body_first
        aliases = {}
    else:
        body = _ln_body
        in_specs.append(pl.BlockSpec(memory_space=pl.ANY))
        args.append(prev)
        aliases = {6: 0}
    return pl.pallas_call(
        body,
        grid=(BSPLIT,),
        in_specs=in_specs,
        out_specs=pl.BlockSpec((1, S, D), lambda b, off=off: (off + b, 0, 0)),
        out_shape=jax.ShapeDtypeStruct((B, S, D), jnp.float32),
        input_output_aliases=aliases,
    )(*args)


def kernel(input_ids, token_type_ids, word_emb, type_emb, pos_emb, gamma, beta):
    ids = input_ids.astype(jnp.int32).reshape(NSPLIT, NW, NCHUNK, CHUNK)
    tt_col = token_type_ids.astype(jnp.float32).reshape(B, S, 1)
    g2 = gamma.reshape(1, D)
    b2 = beta.reshape(1, D)
    ws = [_sc_gather(ids[i], word_emb) for i in range(NSPLIT)]
    out = None
    for i in range(NSPLIT):
        out = _tc_ln_part(ws[i], tt_col, type_emb, pos_emb, g2, b2, out, i)
    return out
